# baseline (speedup 1.0000x reference)
import functools

import jax
import jax.numpy as jnp
from jax import lax
from jax.experimental import pallas as pl
from jax.experimental.pallas import tpu as pltpu
from jax.experimental.pallas import tpu_sc as plsc

F32 = jnp.float32

_NC, _NS = 2, 16
_NW = _NC * _NS
_CH = 128


def _sigmoid(x):
    return 1.0 / (1.0 + jnp.exp(-x))


def _silu(x):
    return x * _sigmoid(x)


def _proj_body(h_ref, wi_ref, wj_ref, a_ref, b_ref):
    h = h_ref[...]
    a_ref[...] = jnp.dot(h, wi_ref[...], preferred_element_type=F32)
    b_ref[...] = jnp.dot(h, wj_ref[...], preferred_element_type=F32)


def _proj(h, wi, wj):
    n, d = h.shape
    blk = 2000
    return pl.pallas_call(
        _proj_body,
        grid=(n // blk,),
        in_specs=[
            pl.BlockSpec((blk, d), lambda i: (i, 0)),
            pl.BlockSpec((d, 64), lambda i: (0, 0)),
            pl.BlockSpec((d, 64), lambda i: (0, 0)),
        ],
        out_specs=[
            pl.BlockSpec((blk, 64), lambda i: (i, 0)),
            pl.BlockSpec((blk, 64), lambda i: (i, 0)),
        ],
        out_shape=[
            jax.ShapeDtypeStruct((n, 64), F32),
            jax.ShapeDtypeStruct((n, 64), F32),
        ],
    )(h, wi, wj)


def _gather_body(e, a_hbm, b_hbm, ii_hbm, jj_hbm, g_hbm,
                 ii_v, jj_v, rows_a, rows_b, sem_a, sem_b):
    c = lax.axis_index("c")
    s = lax.axis_index("s")
    wid = s * _NC + c
    ep = e // _NW
    nfull = ep // _CH
    tail = ep - nfull * _CH
    base = pl.multiple_of(wid * ep, _CH)

    pltpu.sync_copy(ii_hbm.at[pl.ds(base, ep)], ii_v)
    pltpu.sync_copy(jj_hbm.at[pl.ds(base, ep)], jj_v)

    def chunk(k, _):
        off = pl.multiple_of(k * _CH, _CH)
        ca = pltpu.async_copy(a_hbm.at[ii_v.at[pl.ds(off, _CH)]], rows_a, sem_a)
        cb = pltpu.async_copy(b_hbm.at[jj_v.at[pl.ds(off, _CH)]], rows_b, sem_b)
        ca.wait()
        cb.wait()
        pltpu.sync_copy(rows_a, g_hbm.at[pl.ds(base + off, _CH), pl.ds(0, 64)])
        pltpu.sync_copy(rows_b, g_hbm.at[pl.ds(base + off, _CH), pl.ds(64, 64)])
        return 0

    lax.fori_loop(0, nfull, chunk, 0)

    if tail:
        off = nfull * _CH
        ca = pltpu.async_copy(a_hbm.at[ii_v.at[pl.ds(off, tail)]],
                              rows_a.at[pl.ds(0, tail)], sem_a)
        cb = pltpu.async_copy(b_hbm.at[jj_v.at[pl.ds(off, tail)]],
                              rows_b.at[pl.ds(0, tail)], sem_b)
        ca.wait()
        cb.wait()
        pltpu.sync_copy(rows_a.at[pl.ds(0, tail)],
                        g_hbm.at[pl.ds(base + off, tail), pl.ds(0, 64)])
        pltpu.sync_copy(rows_b.at[pl.ds(0, tail)],
                        g_hbm.at[pl.ds(base + off, tail), pl.ds(64, 64)])


def _gather(a, bm, ii, jj):
    n, d = a.shape
    e = ii.shape[0]
    ep = e // _NW
    mesh = plsc.VectorSubcoreMesh(core_axis_name="c", subcore_axis_name="s",
                                  num_cores=_NC, num_subcores=_NS)
    k = pl.kernel(
        functools.partial(_gather_body, e),
        mesh=mesh,
        compiler_params=pltpu.CompilerParams(use_tc_tiling_on_sc=False),
        out_type=jax.ShapeDtypeStruct((e, 2 * d), F32),
        scratch_types=[
            pltpu.VMEM((ep,), jnp.int32),
            pltpu.VMEM((ep,), jnp.int32),
            pltpu.VMEM((_CH, d), F32),
            pltpu.VMEM((_CH, d), F32),
            pltpu.SemaphoreType.DMA,
            pltpu.SemaphoreType.DMA,
        ],
    )
    return k(a, bm, ii, jj)


def _edge_body(g_ref, ft_ref,
               w1g_ref, eb1_ref, ew2_ref, eb2_ref,
               gw1_ref, gb1_ref, gw2_ref, gb2_ref, out_ref):
    g = g_ref[...]
    x = g[:, 0:64] + g[:, 64:128]
    ft = ft_ref[...]
    pre = (x
           + jax.lax.dot_general(ft, w1g_ref[...], (((0,), (0,)), ((), ())),
                                 preferred_element_type=F32,
                                 precision=lax.Precision.HIGHEST)
           + eb1_ref[...])
    h = _silu(pre)
    e = _silu(jnp.dot(h, ew2_ref[...], preferred_element_type=F32) + eb2_ref[...])
    g1 = (jax.lax.dot_general(ft, gw1_ref[...], (((0,), (0,)), ((), ())),
                              preferred_element_type=F32,
                              precision=lax.Precision.HIGHEST)
          + gb1_ref[...])
    gh = _silu(g1)
    glogit = jnp.sum(gh * gw2_ref[...], axis=-1, keepdims=True) + gb2_ref[...]
    msg = e * _sigmoid(glogit)
    out_ref[...] = jnp.concatenate([msg, jnp.zeros_like(msg)], axis=1)


def _edge(g, ft, w1g, eb1, ew2, eb2, gw1, gb1, gw2, gb2):
    e = g.shape[0]
    blk = 6400
    wspec = lambda shape: pl.BlockSpec(shape, lambda i: tuple(0 for _ in shape))
    return pl.pallas_call(
        _edge_body,
        grid=(e // blk,),
        in_specs=[
            pl.BlockSpec((blk, 128), lambda i: (i, 0)),
            pl.BlockSpec((10, blk), lambda i: (0, i)),
            wspec((10, 64)), wspec((1, 64)), wspec((64, 64)), wspec((1, 64)),
            wspec((10, 32)), wspec((1, 32)), wspec((1, 32)), wspec((1, 1)),
        ],
        out_specs=pl.BlockSpec((blk, 128), lambda i: (i, 0)),
        out_shape=jax.ShapeDtypeStruct((e, 128), F32),
    )(g, ft, w1g, eb1, ew2, eb2, gw1, gb1, gw2, gb2)


def _scatter_body(e, n, msg_hbm, ii_hbm, zero_hbm, out_hbm,
                  idx_v, idx_t, rows_v, rows_t, agg_sh):
    c = lax.axis_index("c")
    s = lax.axis_index("s")
    wid = s * _NC + c
    ep = e // _NW
    nfull = ep // _CH
    tail = ep - nfull * _CH
    base = pl.multiple_of(wid * ep, _CH)
    npart = n // _NS

    pltpu.sync_copy(zero_hbm.at[pl.ds(s * npart, npart)],
                    agg_sh.at[pl.ds(s * npart, npart)])
    plsc.subcore_barrier()

    def chunk(k, _):
        off = pl.multiple_of(k * _CH, _CH)
        pltpu.sync_copy(msg_hbm.at[pl.ds(base + off, _CH)], rows_v)
        pltpu.sync_copy(ii_hbm.at[pl.ds(base + off, _CH)], idx_v)
        pltpu.sync_copy(rows_v, agg_sh.at[idx_v], add=True)
        return 0

    lax.fori_loop(0, nfull, chunk, 0)

    if tail:
        off = nfull * _CH
        pltpu.sync_copy(msg_hbm.at[pl.ds(base + off, tail)], rows_t)
        pltpu.sync_copy(ii_hbm.at[pl.ds(base + off, tail)], idx_t)
        pltpu.sync_copy(rows_t, agg_sh.at[idx_t], add=True)

    plsc.subcore_barrier()
    pltpu.sync_copy(agg_sh.at[pl.ds(s * npart, npart)],
                    out_hbm.at[pl.ds(c * n + s * npart, npart)])


def _scatter(msg, ii, zero):
    e, d = msg.shape
    n = zero.shape[0]
    ep = e // _NW
    tail = ep - (ep // _CH) * _CH
    mesh = plsc.VectorSubcoreMesh(core_axis_name="c", subcore_axis_name="s",
                                  num_cores=_NC, num_subcores=_NS)
    k = pl.kernel(
        functools.partial(_scatter_body, e, n),
        mesh=mesh,
        compiler_params=pltpu.CompilerParams(use_tc_tiling_on_sc=False),
        out_type=jax.ShapeDtypeStruct((_NC * n, d), F32),
        scratch_types=[
            pltpu.VMEM((_CH,), jnp.int32),
            pltpu.VMEM((max(tail, 8),), jnp.int32),
            pltpu.VMEM((_CH, d), F32),
            pltpu.VMEM((max(tail, 8), d), F32),
            pltpu.VMEM_SHARED((n, d), F32),
        ],
    )
    return k(msg, ii, zero)


def _node_body(h_ref, agg2_ref, w1a_ref, w1b_ref, nb1_ref, w2_ref, nb2_ref,
               g_ref, b_ref, out_ref):
    h = h_ref[...]
    agg = (agg2_ref[0] + agg2_ref[1])[:, 0:64]
    m1 = (jnp.dot(h, w1a_ref[...], preferred_element_type=F32)
          + jnp.dot(agg, w1b_ref[...], preferred_element_type=F32)
          + nb1_ref[...])
    hm = _silu(m1)
    m = jnp.dot(hm, w2_ref[...], preferred_element_type=F32) + nb2_ref[...]
    y = h + m
    mu = jnp.mean(y, axis=-1, keepdims=True)
    yc = y - mu
    var = jnp.mean(yc * yc, axis=-1, keepdims=True)
    out_ref[...] = yc * lax.rsqrt(var + 1e-5) * g_ref[...] + b_ref[...]


def _node(h, agg2, w1a, w1b, nb1, w2, nb2, g, b):
    n, d = h.shape
    blk = 2000
    wspec = lambda shape: pl.BlockSpec(shape, lambda i: tuple(0 for _ in shape))
    return pl.pallas_call(
        _node_body,
        grid=(n // blk,),
        in_specs=[
            pl.BlockSpec((blk, d), lambda i: (i, 0)),
            pl.BlockSpec((2, blk, 128), lambda i: (0, i, 0)),
            wspec((d, 2 * d)), wspec((64, 2 * d)), wspec((1, 2 * d)),
            wspec((2 * d, d)), wspec((1, d)),
            wspec((1, d)), wspec((1, d)),
        ],
        out_specs=pl.BlockSpec((blk, d), lambda i: (i, 0)),
        out_shape=jax.ShapeDtypeStruct((n, d), F32),
    )(h, agg2, w1a, w1b, nb1, w2, nb2, g, b)


def kernel(H, edge_index, dist2, delta, edge_struct,
           eW1, eb1, eW2, eb2, gW1, gb1, gW2, gb2,
           nW1, nb1, nW2, nb2, ln_g, ln_b):
    bz, n, d = H.shape
    e = edge_index.shape[1]
    d_struct = edge_struct.shape[-1]
    assert bz == 1 and e % _NW == 0 and d == 128

    h0 = H.reshape(n, d)
    ii = edge_index[0]
    jj = edge_index[1]
    ft = jnp.concatenate([dist2.reshape(1, e), delta.reshape(1, e),
                          edge_struct.reshape(e, d_struct).T], axis=0)

    wi = eW1[0:d]
    wj = eW1[d:2 * d]
    w1g = eW1[2 * d:]

    a, bm = _proj(h0, wi, wj)
    g = _gather(a, bm, ii, jj)
    emsg = _edge(g, ft, w1g,
                 eb1.reshape(1, -1), eW2, eb2.reshape(1, -1),
                 gW1, gb1.reshape(1, -1), gW2.reshape(1, -1), gb2.reshape(1, 1))
    zero = jnp.zeros((n, 128), F32)
    agg2 = _scatter(emsg, ii, zero).reshape(_NC, n, 128)
    out = _node(h0, agg2,
                nW1[0:d], nW1[d:], nb1.reshape(1, -1),
                nW2, nb2.reshape(1, -1),
                ln_g.reshape(1, -1), ln_b.reshape(1, -1))
    return out.reshape(bz, n, d)


# speedup vs baseline: 2.1491x; 2.1491x over previous
import functools

import jax
import jax.numpy as jnp
from jax import lax
from jax.experimental import pallas as pl
from jax.experimental.pallas import tpu as pltpu
from jax.experimental.pallas import tpu_sc as plsc

F32 = jnp.float32

_NC, _NS = 2, 16
_NW = _NC * _NS
_CH = 128


def _sigmoid(x):
    return 1.0 / (1.0 + jnp.exp(-x))


def _silu(x):
    return x * _sigmoid(x)


def _proj_body(h_ref, wi_ref, wj_ref, a_ref, b_ref):
    h = h_ref[...]
    a_ref[...] = jnp.dot(h, wi_ref[...], preferred_element_type=F32)
    b_ref[...] = jnp.dot(h, wj_ref[...], preferred_element_type=F32)


def _proj(h, wi, wj):
    n, d = h.shape
    blk = 2000
    return pl.pallas_call(
        _proj_body,
        grid=(n // blk,),
        in_specs=[
            pl.BlockSpec((blk, d), lambda i: (i, 0)),
            pl.BlockSpec((d, 64), lambda i: (0, 0)),
            pl.BlockSpec((d, 64), lambda i: (0, 0)),
        ],
        out_specs=[
            pl.BlockSpec((blk, 64), lambda i: (i, 0)),
            pl.BlockSpec((blk, 64), lambda i: (i, 0)),
        ],
        out_shape=[
            jax.ShapeDtypeStruct((n, 64), F32),
            jax.ShapeDtypeStruct((n, 64), F32),
        ],
    )(h, wi, wj)


def _gather_body(e, a_hbm, b_hbm, ii_hbm, jj_hbm, g_hbm,
                 ii_v, jj_v, rows_a, rows_b, sem_a, sem_b):
    c = lax.axis_index("c")
    s = lax.axis_index("s")
    wid = s * _NC + c
    ep = e // _NW
    nfull = ep // _CH
    tail = ep - nfull * _CH
    base = pl.multiple_of(wid * ep, _CH)

    pltpu.sync_copy(ii_hbm.at[pl.ds(base, ep)], ii_v)
    pltpu.sync_copy(jj_hbm.at[pl.ds(base, ep)], jj_v)

    def chunk(k, _):
        off = pl.multiple_of(k * _CH, _CH)
        ca = pltpu.async_copy(a_hbm.at[ii_v.at[pl.ds(off, _CH)]], rows_a, sem_a)
        cb = pltpu.async_copy(b_hbm.at[jj_v.at[pl.ds(off, _CH)]], rows_b, sem_b)
        ca.wait()
        cb.wait()
        pltpu.sync_copy(rows_a, g_hbm.at[pl.ds(base + off, _CH), pl.ds(0, 64)])
        pltpu.sync_copy(rows_b, g_hbm.at[pl.ds(base + off, _CH), pl.ds(64, 64)])
        return 0

    lax.fori_loop(0, nfull, chunk, 0)

    if tail:
        off = nfull * _CH
        ca = pltpu.async_copy(a_hbm.at[ii_v.at[pl.ds(off, tail)]],
                              rows_a.at[pl.ds(0, tail)], sem_a)
        cb = pltpu.async_copy(b_hbm.at[jj_v.at[pl.ds(off, tail)]],
                              rows_b.at[pl.ds(0, tail)], sem_b)
        ca.wait()
        cb.wait()
        pltpu.sync_copy(rows_a.at[pl.ds(0, tail)],
                        g_hbm.at[pl.ds(base + off, tail), pl.ds(0, 64)])
        pltpu.sync_copy(rows_b.at[pl.ds(0, tail)],
                        g_hbm.at[pl.ds(base + off, tail), pl.ds(64, 64)])


def _gather(a, bm, ii, jj):
    n, d = a.shape
    e = ii.shape[0]
    ep = e // _NW
    mesh = plsc.VectorSubcoreMesh(core_axis_name="c", subcore_axis_name="s",
                                  num_cores=_NC, num_subcores=_NS)
    k = pl.kernel(
        functools.partial(_gather_body, e),
        mesh=mesh,
        compiler_params=pltpu.CompilerParams(use_tc_tiling_on_sc=False),
        out_type=jax.ShapeDtypeStruct((e, 2 * d), F32),
        scratch_types=[
            pltpu.VMEM((ep,), jnp.int32),
            pltpu.VMEM((ep,), jnp.int32),
            pltpu.VMEM((_CH, d), F32),
            pltpu.VMEM((_CH, d), F32),
            pltpu.SemaphoreType.DMA,
            pltpu.SemaphoreType.DMA,
        ],
    )
    return k(a, bm, ii, jj)


def _edge_body(g_ref, ft_ref,
               w1g_ref, eb1_ref, ew2_ref, eb2_ref,
               gw1_ref, gb1_ref, gw2_ref, gb2_ref, out_ref):
    g = g_ref[...]
    x = g[:, 0:64] + g[:, 64:128]
    ft = ft_ref[...]
    pre = (x
           + jax.lax.dot_general(ft, w1g_ref[...], (((0,), (0,)), ((), ())),
                                 preferred_element_type=F32)
           + eb1_ref[...])
    h = _silu(pre)
    e = _silu(jnp.dot(h, ew2_ref[...], preferred_element_type=F32) + eb2_ref[...])
    g1 = (jax.lax.dot_general(ft, gw1_ref[...], (((0,), (0,)), ((), ())),
                              preferred_element_type=F32)
          + gb1_ref[...])
    gh = _silu(g1)
    glogit = jnp.sum(gh * gw2_ref[...], axis=-1, keepdims=True) + gb2_ref[...]
    msg = e * _sigmoid(glogit)
    out_ref[...] = jnp.concatenate([msg, jnp.zeros_like(msg)], axis=1)


def _edge(g, ft, w1g, eb1, ew2, eb2, gw1, gb1, gw2, gb2):
    e = g.shape[0]
    blk = 6400
    wspec = lambda shape: pl.BlockSpec(shape, lambda i: tuple(0 for _ in shape))
    return pl.pallas_call(
        _edge_body,
        grid=(e // blk,),
        in_specs=[
            pl.BlockSpec((blk, 128), lambda i: (i, 0)),
            pl.BlockSpec((10, blk), lambda i: (0, i)),
            wspec((10, 64)), wspec((1, 64)), wspec((64, 64)), wspec((1, 64)),
            wspec((10, 32)), wspec((1, 32)), wspec((1, 32)), wspec((1, 1)),
        ],
        out_specs=pl.BlockSpec((blk, 128), lambda i: (i, 0)),
        out_shape=jax.ShapeDtypeStruct((e, 128), F32),
    )(g, ft, w1g, eb1, ew2, eb2, gw1, gb1, gw2, gb2)


def _scatter_body(e, n, msg_hbm, ii_hbm, zero_hbm, out_hbm,
                  idx_v, idx_t, rows_v, rows_t, agg_sh):
    c = lax.axis_index("c")
    s = lax.axis_index("s")
    wid = s * _NC + c
    ep = e // _NW
    nfull = ep // _CH
    tail = ep - nfull * _CH
    base = pl.multiple_of(wid * ep, _CH)
    npart = n // _NS

    pltpu.sync_copy(zero_hbm.at[pl.ds(s * npart, npart)],
                    agg_sh.at[pl.ds(s * npart, npart)])
    plsc.subcore_barrier()

    def chunk(k, _):
        off = pl.multiple_of(k * _CH, _CH)
        pltpu.sync_copy(msg_hbm.at[pl.ds(base + off, _CH)], rows_v)
        pltpu.sync_copy(ii_hbm.at[pl.ds(base + off, _CH)], idx_v)
        pltpu.sync_copy(rows_v, agg_sh.at[idx_v], add=True)
        return 0

    lax.fori_loop(0, nfull, chunk, 0)

    if tail:
        off = nfull * _CH
        pltpu.sync_copy(msg_hbm.at[pl.ds(base + off, tail)], rows_t)
        pltpu.sync_copy(ii_hbm.at[pl.ds(base + off, tail)], idx_t)
        pltpu.sync_copy(rows_t, agg_sh.at[idx_t], add=True)

    plsc.subcore_barrier()
    pltpu.sync_copy(agg_sh.at[pl.ds(s * npart, npart)],
                    out_hbm.at[pl.ds(c * n + s * npart, npart)])


def _scatter(msg, ii, zero):
    e, d = msg.shape
    n = zero.shape[0]
    ep = e // _NW
    tail = ep - (ep // _CH) * _CH
    mesh = plsc.VectorSubcoreMesh(core_axis_name="c", subcore_axis_name="s",
                                  num_cores=_NC, num_subcores=_NS)
    k = pl.kernel(
        functools.partial(_scatter_body, e, n),
        mesh=mesh,
        compiler_params=pltpu.CompilerParams(use_tc_tiling_on_sc=False),
        out_type=jax.ShapeDtypeStruct((_NC * n, d), F32),
        scratch_types=[
            pltpu.VMEM((_CH,), jnp.int32),
            pltpu.VMEM((max(tail, 8),), jnp.int32),
            pltpu.VMEM((_CH, d), F32),
            pltpu.VMEM((max(tail, 8), d), F32),
            pltpu.VMEM_SHARED((n, d), F32),
        ],
    )
    return k(msg, ii, zero)


def _node_body(h_ref, agg2_ref, w1a_ref, w1b_ref, nb1_ref, w2_ref, nb2_ref,
               g_ref, b_ref, out_ref):
    h = h_ref[...]
    agg = (agg2_ref[0] + agg2_ref[1])[:, 0:64]
    m1 = (jnp.dot(h, w1a_ref[...], preferred_element_type=F32)
          + jnp.dot(agg, w1b_ref[...], preferred_element_type=F32)
          + nb1_ref[...])
    hm = _silu(m1)
    m = jnp.dot(hm, w2_ref[...], preferred_element_type=F32) + nb2_ref[...]
    y = h + m
    mu = jnp.mean(y, axis=-1, keepdims=True)
    yc = y - mu
    var = jnp.mean(yc * yc, axis=-1, keepdims=True)
    out_ref[...] = yc * lax.rsqrt(var + 1e-5) * g_ref[...] + b_ref[...]


def _node(h, agg2, w1a, w1b, nb1, w2, nb2, g, b):
    n, d = h.shape
    blk = 2000
    wspec = lambda shape: pl.BlockSpec(shape, lambda i: tuple(0 for _ in shape))
    return pl.pallas_call(
        _node_body,
        grid=(n // blk,),
        in_specs=[
            pl.BlockSpec((blk, d), lambda i: (i, 0)),
            pl.BlockSpec((2, blk, 128), lambda i: (0, i, 0)),
            wspec((d, 2 * d)), wspec((64, 2 * d)), wspec((1, 2 * d)),
            wspec((2 * d, d)), wspec((1, d)),
            wspec((1, d)), wspec((1, d)),
        ],
        out_specs=pl.BlockSpec((blk, d), lambda i: (i, 0)),
        out_shape=jax.ShapeDtypeStruct((n, d), F32),
    )(h, agg2, w1a, w1b, nb1, w2, nb2, g, b)


def kernel(H, edge_index, dist2, delta, edge_struct,
           eW1, eb1, eW2, eb2, gW1, gb1, gW2, gb2,
           nW1, nb1, nW2, nb2, ln_g, ln_b):
    bz, n, d = H.shape
    e = edge_index.shape[1]
    d_struct = edge_struct.shape[-1]
    assert bz == 1 and e % _NW == 0 and d == 128

    h0 = H.reshape(n, d)
    ii = edge_index[0]
    jj = edge_index[1]
    ft = jnp.concatenate([dist2.reshape(1, e), delta.reshape(1, e),
                          edge_struct.reshape(e, d_struct).T], axis=0)

    wi = eW1[0:d]
    wj = eW1[d:2 * d]
    w1g = eW1[2 * d:]

    a, bm = _proj(h0, wi, wj)
    g = _gather(a, bm, ii, jj)
    emsg = _edge(g, ft, w1g,
                 eb1.reshape(1, -1), eW2, eb2.reshape(1, -1),
                 gW1, gb1.reshape(1, -1), gW2.reshape(1, -1), gb2.reshape(1, 1))
    zero = jnp.zeros((n, 128), F32)
    agg2 = _scatter(emsg, ii, zero).reshape(_NC, n, 128)
    out = _node(h0, agg2,
                nW1[0:d], nW1[d:], nb1.reshape(1, -1),
                nW2, nb2.reshape(1, -1),
                ln_g.reshape(1, -1), ln_b.reshape(1, -1))
    return out.reshape(bz, n, d)


# speedup vs baseline: 2.9179x; 1.3577x over previous
import functools

import jax
import jax.numpy as jnp
from jax import lax
from jax.experimental import pallas as pl
from jax.experimental.pallas import tpu as pltpu
from jax.experimental.pallas import tpu_sc as plsc

F32 = jnp.float32

_NC, _NS = 2, 16
_NW = _NC * _NS
_CH = 128


def _sigmoid(x):
    return 1.0 / (1.0 + jnp.exp(-x))


def _silu(x):
    return x * _sigmoid(x)


def _proj_body(h_ref, wi_ref, wj_ref, a_ref, b_ref):
    h = h_ref[...]
    a_ref[...] = jnp.dot(h, wi_ref[...], preferred_element_type=F32)
    b_ref[...] = jnp.dot(h, wj_ref[...], preferred_element_type=F32)


def _proj(h, wi, wj):
    n, d = h.shape
    blk = 2000
    return pl.pallas_call(
        _proj_body,
        grid=(n // blk,),
        in_specs=[
            pl.BlockSpec((blk, d), lambda i: (i, 0)),
            pl.BlockSpec((d, 64), lambda i: (0, 0)),
            pl.BlockSpec((d, 64), lambda i: (0, 0)),
        ],
        out_specs=[
            pl.BlockSpec((blk, 64), lambda i: (i, 0)),
            pl.BlockSpec((blk, 64), lambda i: (i, 0)),
        ],
        out_shape=[
            jax.ShapeDtypeStruct((n, 64), F32),
            jax.ShapeDtypeStruct((n, 64), F32),
        ],
    )(h, wi, wj)


def _gather_body(e, a_hbm, b_hbm, ii_hbm, jj_hbm, g_hbm,
                 ii_v, jj_v, rows_a, rows_b, sga0, sga1, sgb0, sgb1):
    c = lax.axis_index("c")
    s = lax.axis_index("s")
    wid = s * _NC + c
    ep = e // _NW
    nfull = ep // _CH
    tail = ep - nfull * _CH
    base = pl.multiple_of(wid * ep, _CH)
    sga = (sga0, sga1)
    sgb = (sgb0, sgb1)

    pltpu.sync_copy(ii_hbm.at[pl.ds(base, ep)], ii_v)
    pltpu.sync_copy(jj_hbm.at[pl.ds(base, ep)], jj_v)

    def fire(cidx, b):
        off = pl.multiple_of(cidx * _CH, _CH)
        pltpu.async_copy(a_hbm.at[ii_v.at[pl.ds(off, _CH)]], rows_a.at[b], sga[b])
        pltpu.async_copy(b_hbm.at[jj_v.at[pl.ds(off, _CH)]], rows_b.at[b], sgb[b])

    for b in range(2):
        fire(b, b)

    def pair(p, _):
        for b in range(2):
            cidx = 2 * p + b
            off = pl.multiple_of(cidx * _CH, _CH)
            pltpu.make_async_copy(a_hbm.at[ii_v.at[pl.ds(off, _CH)]],
                                  rows_a.at[b], sga[b]).wait()
            pltpu.make_async_copy(b_hbm.at[jj_v.at[pl.ds(off, _CH)]],
                                  rows_b.at[b], sgb[b]).wait()
            pltpu.sync_copy(rows_a.at[b],
                            g_hbm.at[pl.ds(base + off, _CH), pl.ds(0, 64)])
            pltpu.sync_copy(rows_b.at[b],
                            g_hbm.at[pl.ds(base + off, _CH), pl.ds(64, 64)])

            @pl.when(cidx + 2 < nfull)
            def _():
                fire(cidx + 2, b)
        return 0

    lax.fori_loop(0, nfull // 2, pair, 0)

    if tail:
        off = nfull * _CH
        ca = pltpu.async_copy(a_hbm.at[ii_v.at[pl.ds(off, tail)]],
                              rows_a.at[0, pl.ds(0, tail)], sga0)
        cb = pltpu.async_copy(b_hbm.at[jj_v.at[pl.ds(off, tail)]],
                              rows_b.at[0, pl.ds(0, tail)], sgb0)
        ca.wait()
        cb.wait()
        pltpu.sync_copy(rows_a.at[0, pl.ds(0, tail)],
                        g_hbm.at[pl.ds(base + off, tail), pl.ds(0, 64)])
        pltpu.sync_copy(rows_b.at[0, pl.ds(0, tail)],
                        g_hbm.at[pl.ds(base + off, tail), pl.ds(64, 64)])


def _gather(a, bm, ii, jj):
    n, d = a.shape
    e = ii.shape[0]
    ep = e // _NW
    mesh = plsc.VectorSubcoreMesh(core_axis_name="c", subcore_axis_name="s",
                                  num_cores=_NC, num_subcores=_NS)
    k = pl.kernel(
        functools.partial(_gather_body, e),
        mesh=mesh,
        compiler_params=pltpu.CompilerParams(use_tc_tiling_on_sc=False),
        out_type=jax.ShapeDtypeStruct((e, 2 * d), F32),
        scratch_types=[
            pltpu.VMEM((ep,), jnp.int32),
            pltpu.VMEM((ep,), jnp.int32),
            pltpu.VMEM((2, _CH, d), F32),
            pltpu.VMEM((2, _CH, d), F32),
            pltpu.SemaphoreType.DMA,
            pltpu.SemaphoreType.DMA,
            pltpu.SemaphoreType.DMA,
            pltpu.SemaphoreType.DMA,
        ],
    )
    return k(a, bm, ii, jj)


def _edge_body(g_ref, ft_ref,
               w1g_ref, eb1_ref, ew2_ref, eb2_ref,
               gw1_ref, gb1_ref, gw2_ref, gb2_ref, out_ref):
    g = g_ref[...]
    x = g[:, 0:64] + g[:, 64:128]
    ft = ft_ref[...]
    pre = (x
           + jax.lax.dot_general(ft, w1g_ref[...], (((0,), (0,)), ((), ())),
                                 preferred_element_type=F32)
           + eb1_ref[...])
    h = _silu(pre)
    e = _silu(jnp.dot(h, ew2_ref[...], preferred_element_type=F32) + eb2_ref[...])
    g1 = (jax.lax.dot_general(ft, gw1_ref[...], (((0,), (0,)), ((), ())),
                              preferred_element_type=F32)
          + gb1_ref[...])
    gh = _silu(g1)
    glogit = jnp.sum(gh * gw2_ref[...], axis=-1, keepdims=True) + gb2_ref[...]
    msg = e * _sigmoid(glogit)
    out_ref[...] = jnp.concatenate([msg, jnp.zeros_like(msg)], axis=1)


def _edge(g, ft, w1g, eb1, ew2, eb2, gw1, gb1, gw2, gb2):
    e = g.shape[0]
    blk = 6400
    wspec = lambda shape: pl.BlockSpec(shape, lambda i: tuple(0 for _ in shape))
    return pl.pallas_call(
        _edge_body,
        grid=(e // blk,),
        in_specs=[
            pl.BlockSpec((blk, 128), lambda i: (i, 0)),
            pl.BlockSpec((10, blk), lambda i: (0, i)),
            wspec((10, 64)), wspec((1, 64)), wspec((64, 64)), wspec((1, 64)),
            wspec((10, 32)), wspec((1, 32)), wspec((1, 32)), wspec((1, 1)),
        ],
        out_specs=pl.BlockSpec((blk, 128), lambda i: (i, 0)),
        out_shape=jax.ShapeDtypeStruct((e, 128), F32),
    )(g, ft, w1g, eb1, ew2, eb2, gw1, gb1, gw2, gb2)


def _scatter_body(e, n, msg_hbm, ii_hbm, zero_hbm, out_hbm,
                  idx_v, idx_t, rows_v, rows_t, agg_sh, sr0, sr1, si0, si1):
    c = lax.axis_index("c")
    s = lax.axis_index("s")
    wid = s * _NC + c
    ep = e // _NW
    nfull = ep // _CH
    tail = ep - nfull * _CH
    base = pl.multiple_of(wid * ep, _CH)
    npart = n // _NS
    sr = (sr0, sr1)
    si = (si0, si1)

    pltpu.sync_copy(zero_hbm.at[pl.ds(s * npart, npart)],
                    agg_sh.at[pl.ds(s * npart, npart)])
    plsc.subcore_barrier()

    def fire(cidx, b):
        off = pl.multiple_of(cidx * _CH, _CH)
        pltpu.async_copy(msg_hbm.at[pl.ds(base + off, _CH), pl.ds(0, 64)],
                         rows_v.at[b], sr[b])
        pltpu.async_copy(ii_hbm.at[pl.ds(base + off, _CH)], idx_v.at[b], si[b])

    for b in range(2):
        fire(b, b)

    def pair(p, _):
        for b in range(2):
            cidx = 2 * p + b
            off = pl.multiple_of(cidx * _CH, _CH)
            pltpu.make_async_copy(msg_hbm.at[pl.ds(base + off, _CH), pl.ds(0, 64)],
                                  rows_v.at[b], sr[b]).wait()
            pltpu.make_async_copy(ii_hbm.at[pl.ds(base + off, _CH)],
                                  idx_v.at[b], si[b]).wait()
            pltpu.sync_copy(rows_v.at[b], agg_sh.at[idx_v.at[b]], add=True)

            @pl.when(cidx + 2 < nfull)
            def _():
                fire(cidx + 2, b)
        return 0

    lax.fori_loop(0, nfull // 2, pair, 0)

    if tail:
        off = nfull * _CH
        pltpu.sync_copy(msg_hbm.at[pl.ds(base + off, tail), pl.ds(0, 64)], rows_t)
        pltpu.sync_copy(ii_hbm.at[pl.ds(base + off, tail)], idx_t)
        pltpu.sync_copy(rows_t, agg_sh.at[idx_t], add=True)

    plsc.subcore_barrier()
    pltpu.sync_copy(agg_sh.at[pl.ds(s * npart, npart)],
                    out_hbm.at[pl.ds(c * n + s * npart, npart), pl.ds(0, 64)])


def _scatter(msg, ii, zero):
    e = msg.shape[0]
    n = zero.shape[0]
    ep = e // _NW
    tail = ep - (ep // _CH) * _CH
    mesh = plsc.VectorSubcoreMesh(core_axis_name="c", subcore_axis_name="s",
                                  num_cores=_NC, num_subcores=_NS)
    k = pl.kernel(
        functools.partial(_scatter_body, e, n),
        mesh=mesh,
        compiler_params=pltpu.CompilerParams(use_tc_tiling_on_sc=False),
        out_type=jax.ShapeDtypeStruct((_NC * n, 128), F32),
        scratch_types=[
            pltpu.VMEM((2, _CH), jnp.int32),
            pltpu.VMEM((max(tail, 8),), jnp.int32),
            pltpu.VMEM((2, _CH, 64), F32),
            pltpu.VMEM((max(tail, 8), 64), F32),
            pltpu.VMEM_SHARED((n, 64), F32),
            pltpu.SemaphoreType.DMA,
            pltpu.SemaphoreType.DMA,
            pltpu.SemaphoreType.DMA,
            pltpu.SemaphoreType.DMA,
        ],
    )
    return k(msg, ii, zero)


def _node_body(h_ref, agg2_ref, w1a_ref, w1b_ref, nb1_ref, w2_ref, nb2_ref,
               g_ref, b_ref, out_ref):
    h = h_ref[...]
    agg = (agg2_ref[0] + agg2_ref[1])[:, 0:64]
    m1 = (jnp.dot(h, w1a_ref[...], preferred_element_type=F32)
          + jnp.dot(agg, w1b_ref[...], preferred_element_type=F32)
          + nb1_ref[...])
    hm = _silu(m1)
    m = jnp.dot(hm, w2_ref[...], preferred_element_type=F32) + nb2_ref[...]
    y = h + m
    mu = jnp.mean(y, axis=-1, keepdims=True)
    yc = y - mu
    var = jnp.mean(yc * yc, axis=-1, keepdims=True)
    out_ref[...] = yc * lax.rsqrt(var + 1e-5) * g_ref[...] + b_ref[...]


def _node(h, agg2, w1a, w1b, nb1, w2, nb2, g, b):
    n, d = h.shape
    blk = 2000
    wspec = lambda shape: pl.BlockSpec(shape, lambda i: tuple(0 for _ in shape))
    return pl.pallas_call(
        _node_body,
        grid=(n // blk,),
        in_specs=[
            pl.BlockSpec((blk, d), lambda i: (i, 0)),
            pl.BlockSpec((2, blk, 128), lambda i: (0, i, 0)),
            wspec((d, 2 * d)), wspec((64, 2 * d)), wspec((1, 2 * d)),
            wspec((2 * d, d)), wspec((1, d)),
            wspec((1, d)), wspec((1, d)),
        ],
        out_specs=pl.BlockSpec((blk, d), lambda i: (i, 0)),
        out_shape=jax.ShapeDtypeStruct((n, d), F32),
    )(h, agg2, w1a, w1b, nb1, w2, nb2, g, b)


def kernel(H, edge_index, dist2, delta, edge_struct,
           eW1, eb1, eW2, eb2, gW1, gb1, gW2, gb2,
           nW1, nb1, nW2, nb2, ln_g, ln_b):
    bz, n, d = H.shape
    e = edge_index.shape[1]
    d_struct = edge_struct.shape[-1]
    assert bz == 1 and e % _NW == 0 and d == 128

    h0 = H.reshape(n, d)
    ii = edge_index[0]
    jj = edge_index[1]
    ft = jnp.concatenate([dist2.reshape(1, e), delta.reshape(1, e),
                          edge_struct.reshape(e, d_struct).T], axis=0)

    wi = eW1[0:d]
    wj = eW1[d:2 * d]
    w1g = eW1[2 * d:]

    a, bm = _proj(h0, wi, wj)
    g = _gather(a, bm, ii, jj)
    emsg = _edge(g, ft, w1g,
                 eb1.reshape(1, -1), eW2, eb2.reshape(1, -1),
                 gW1, gb1.reshape(1, -1), gW2.reshape(1, -1), gb2.reshape(1, 1))
    zero = jnp.zeros((n, 64), F32)
    agg2 = _scatter(emsg, ii, zero).reshape(_NC, n, 128)
    out = _node(h0, agg2,
                nW1[0:d], nW1[d:], nb1.reshape(1, -1),
                nW2, nb2.reshape(1, -1),
                ln_g.reshape(1, -1), ln_b.reshape(1, -1))
    return out.reshape(bz, n, d)


# speedup vs baseline: 3.1408x; 1.0764x over previous
import functools

import jax
import jax.numpy as jnp
from jax import lax
from jax.experimental import pallas as pl
from jax.experimental.pallas import tpu as pltpu
from jax.experimental.pallas import tpu_sc as plsc

F32 = jnp.float32

_NC, _NS = 2, 16
_NW = _NC * _NS
_CH = 128


def _sigmoid(x):
    return 1.0 / (1.0 + jnp.exp(-x))


def _silu(x):
    return x * _sigmoid(x)


def _proj_body(h_ref, wi_ref, wj_ref, a_ref, b_ref):
    h = h_ref[...]
    a_ref[...] = jnp.dot(h, wi_ref[...], preferred_element_type=F32)
    b_ref[...] = jnp.dot(h, wj_ref[...], preferred_element_type=F32)


def _proj(h, wi, wj):
    n, d = h.shape
    blk = 2000
    return pl.pallas_call(
        _proj_body,
        grid=(n // blk,),
        in_specs=[
            pl.BlockSpec((blk, d), lambda i: (i, 0)),
            pl.BlockSpec((d, 64), lambda i: (0, 0)),
            pl.BlockSpec((d, 64), lambda i: (0, 0)),
        ],
        out_specs=[
            pl.BlockSpec((blk, 64), lambda i: (i, 0)),
            pl.BlockSpec((blk, 64), lambda i: (i, 0)),
        ],
        out_shape=[
            jax.ShapeDtypeStruct((n, 64), F32),
            jax.ShapeDtypeStruct((n, 64), F32),
        ],
    )(h, wi, wj)


def _pipe_loop(nfull, fire, wait, proc):
    for b in range(min(2, nfull)):
        fire(b, b)

    def pair(p, _):
        for b in range(2):
            cidx = 2 * p + b
            wait(cidx, b)
            proc(cidx, b)

            @pl.when(cidx + 2 < nfull)
            def _():
                fire(cidx + 2, b)
        return 0

    lax.fori_loop(0, nfull // 2, pair, 0)
    if nfull % 2:
        cidx = nfull - 1
        wait(cidx, cidx % 2)
        proc(cidx, cidx % 2)


def _gather_body(e, a_hbm, b_hbm, ii_hbm, jj_hbm, g_hbm,
                 ii_v, jj_v, rows_a, rows_b, sga0, sga1, sgb0, sgb1):
    c = lax.axis_index("c")
    s = lax.axis_index("s")
    wid = s * _NC + c
    ep = e // _NW
    nfull = ep // _CH
    tail = ep - nfull * _CH
    base = pl.multiple_of(wid * ep, _CH)
    sga = (sga0, sga1)
    sgb = (sgb0, sgb1)

    pltpu.sync_copy(ii_hbm.at[pl.ds(base, ep)], ii_v)
    pltpu.sync_copy(jj_hbm.at[pl.ds(base, ep)], jj_v)

    def fire(cidx, b):
        off = pl.multiple_of(cidx * _CH, _CH)
        pltpu.async_copy(a_hbm.at[ii_v.at[pl.ds(off, _CH)]], rows_a.at[b], sga[b])
        pltpu.async_copy(b_hbm.at[jj_v.at[pl.ds(off, _CH)]], rows_b.at[b], sgb[b])

    def wait(cidx, b):
        off = pl.multiple_of(cidx * _CH, _CH)
        pltpu.make_async_copy(a_hbm.at[ii_v.at[pl.ds(off, _CH)]],
                              rows_a.at[b], sga[b]).wait()
        pltpu.make_async_copy(b_hbm.at[jj_v.at[pl.ds(off, _CH)]],
                              rows_b.at[b], sgb[b]).wait()

    def proc(cidx, b):
        off = pl.multiple_of(cidx * _CH, _CH)
        pltpu.sync_copy(rows_a.at[b],
                        g_hbm.at[pl.ds(base + off, _CH), pl.ds(0, 64)])
        pltpu.sync_copy(rows_b.at[b],
                        g_hbm.at[pl.ds(base + off, _CH), pl.ds(64, 64)])

    _pipe_loop(nfull, fire, wait, proc)

    if tail:
        off = nfull * _CH
        ca = pltpu.async_copy(a_hbm.at[ii_v.at[pl.ds(off, tail)]],
                              rows_a.at[0, pl.ds(0, tail)], sga0)
        cb = pltpu.async_copy(b_hbm.at[jj_v.at[pl.ds(off, tail)]],
                              rows_b.at[0, pl.ds(0, tail)], sgb0)
        ca.wait()
        cb.wait()
        pltpu.sync_copy(rows_a.at[0, pl.ds(0, tail)],
                        g_hbm.at[pl.ds(base + off, tail), pl.ds(0, 64)])
        pltpu.sync_copy(rows_b.at[0, pl.ds(0, tail)],
                        g_hbm.at[pl.ds(base + off, tail), pl.ds(64, 64)])


def _gather(a, bm, ii, jj):
    n, d = a.shape
    e = ii.shape[0]
    ep = e // _NW
    mesh = plsc.VectorSubcoreMesh(core_axis_name="c", subcore_axis_name="s",
                                  num_cores=_NC, num_subcores=_NS)
    k = pl.kernel(
        functools.partial(_gather_body, e),
        mesh=mesh,
        compiler_params=pltpu.CompilerParams(use_tc_tiling_on_sc=False),
        out_type=jax.ShapeDtypeStruct((e, 2 * d), F32),
        scratch_types=[
            pltpu.VMEM((ep,), jnp.int32),
            pltpu.VMEM((ep,), jnp.int32),
            pltpu.VMEM((2, _CH, d), F32),
            pltpu.VMEM((2, _CH, d), F32),
            pltpu.SemaphoreType.DMA,
            pltpu.SemaphoreType.DMA,
            pltpu.SemaphoreType.DMA,
            pltpu.SemaphoreType.DMA,
        ],
    )
    return k(a, bm, ii, jj)


def _edge_body(g_ref, ft_ref,
               w1g_ref, eb1_ref, ew2_ref, eb2_ref,
               gw1_ref, gb1_ref, gw2_ref, gb2_ref, out_ref):
    g = g_ref[...]
    x = g[:, 0:64] + g[:, 64:128]
    ft = ft_ref[...]
    pre = (x
           + jax.lax.dot_general(ft, w1g_ref[...], (((0,), (0,)), ((), ())),
                                 preferred_element_type=F32)
           + eb1_ref[...])
    h = _silu(pre)
    e = _silu(jnp.dot(h, ew2_ref[...], preferred_element_type=F32) + eb2_ref[...])
    g1 = (jax.lax.dot_general(ft, gw1_ref[...], (((0,), (0,)), ((), ())),
                              preferred_element_type=F32)
          + gb1_ref[...])
    gh = _silu(g1)
    glogit = jnp.sum(gh * gw2_ref[...], axis=-1, keepdims=True) + gb2_ref[...]
    msg = e * _sigmoid(glogit)
    out_ref[...] = jnp.concatenate([msg, jnp.zeros_like(msg)], axis=1)


def _edge(g, ft, ft_off, w1g, eb1, ew2, eb2, gw1, gb1, gw2, gb2):
    e = g.shape[0]
    blk = 6400
    off_blocks = ft_off // blk
    wspec = lambda shape: pl.BlockSpec(shape, lambda i: tuple(0 for _ in shape))
    return pl.pallas_call(
        _edge_body,
        grid=(e // blk,),
        in_specs=[
            pl.BlockSpec((blk, 128), lambda i: (i, 0)),
            pl.BlockSpec((10, blk), lambda i: (0, i + off_blocks)),
            wspec((10, 64)), wspec((1, 64)), wspec((64, 64)), wspec((1, 64)),
            wspec((10, 32)), wspec((1, 32)), wspec((1, 32)), wspec((1, 1)),
        ],
        out_specs=pl.BlockSpec((blk, 128), lambda i: (i, 0)),
        out_shape=jax.ShapeDtypeStruct((e, 128), F32),
    )(g, ft, w1g, eb1, ew2, eb2, gw1, gb1, gw2, gb2)


def _scatter_body(np_, e, n, *refs):
    msgs = refs[0:np_]
    iis = refs[np_:2 * np_]
    zero_hbm, out_hbm = refs[2 * np_], refs[2 * np_ + 1]
    idx_v, idx_t, rows_v, rows_t, agg_sh, sr0, sr1, si0, si1 = refs[2 * np_ + 2:]
    c = lax.axis_index("c")
    s = lax.axis_index("s")
    wid = s * _NC + c
    ep = e // _NW
    nfull = ep // _CH
    tail = ep - nfull * _CH
    base = pl.multiple_of(wid * ep, _CH)
    npart = n // _NS
    sr = (sr0, sr1)
    si = (si0, si1)

    pltpu.sync_copy(zero_hbm.at[pl.ds(s * npart, npart)],
                    agg_sh.at[pl.ds(s * npart, npart)])
    plsc.subcore_barrier()

    for msg_hbm, ii_hbm in zip(msgs, iis):
        def fire(cidx, b, m=msg_hbm, ix=ii_hbm):
            off = pl.multiple_of(cidx * _CH, _CH)
            pltpu.async_copy(m.at[pl.ds(base + off, _CH), pl.ds(0, 64)],
                             rows_v.at[b], sr[b])
            pltpu.async_copy(ix.at[pl.ds(base + off, _CH)], idx_v.at[b], si[b])

        def wait(cidx, b, m=msg_hbm, ix=ii_hbm):
            off = pl.multiple_of(cidx * _CH, _CH)
            pltpu.make_async_copy(m.at[pl.ds(base + off, _CH), pl.ds(0, 64)],
                                  rows_v.at[b], sr[b]).wait()
            pltpu.make_async_copy(ix.at[pl.ds(base + off, _CH)],
                                  idx_v.at[b], si[b]).wait()

        def proc(cidx, b):
            pltpu.sync_copy(rows_v.at[b], agg_sh.at[idx_v.at[b]], add=True)

        _pipe_loop(nfull, fire, wait, proc)

        if tail:
            off = nfull * _CH
            pltpu.sync_copy(msg_hbm.at[pl.ds(base + off, tail), pl.ds(0, 64)],
                            rows_t)
            pltpu.sync_copy(ii_hbm.at[pl.ds(base + off, tail)], idx_t)
            pltpu.sync_copy(rows_t, agg_sh.at[idx_t], add=True)

    plsc.subcore_barrier()
    pltpu.sync_copy(agg_sh.at[pl.ds(s * npart, npart)],
                    out_hbm.at[pl.ds(c * n + s * npart, npart), pl.ds(0, 64)])


def _scatter(msgs, iis, zero):
    e = msgs[0].shape[0]
    n = zero.shape[0]
    ep = e // _NW
    tail = ep - (ep // _CH) * _CH
    mesh = plsc.VectorSubcoreMesh(core_axis_name="c", subcore_axis_name="s",
                                  num_cores=_NC, num_subcores=_NS)
    k = pl.kernel(
        functools.partial(_scatter_body, len(msgs), e, n),
        mesh=mesh,
        compiler_params=pltpu.CompilerParams(use_tc_tiling_on_sc=False),
        out_type=jax.ShapeDtypeStruct((_NC * n, 128), F32),
        scratch_types=[
            pltpu.VMEM((2, _CH), jnp.int32),
            pltpu.VMEM((max(tail, 8),), jnp.int32),
            pltpu.VMEM((2, _CH, 64), F32),
            pltpu.VMEM((max(tail, 8), 64), F32),
            pltpu.VMEM_SHARED((n, 64), F32),
            pltpu.SemaphoreType.DMA,
            pltpu.SemaphoreType.DMA,
            pltpu.SemaphoreType.DMA,
            pltpu.SemaphoreType.DMA,
        ],
    )
    return k(*msgs, *iis, zero)


def _node_body(h_ref, agg2_ref, w1a_ref, w1b_ref, nb1_ref, w2_ref, nb2_ref,
               g_ref, b_ref, out_ref):
    h = h_ref[...]
    agg = (agg2_ref[0] + agg2_ref[1])[:, 0:64]
    m1 = (jnp.dot(h, w1a_ref[...], preferred_element_type=F32)
          + jnp.dot(agg, w1b_ref[...], preferred_element_type=F32)
          + nb1_ref[...])
    hm = _silu(m1)
    m = jnp.dot(hm, w2_ref[...], preferred_element_type=F32) + nb2_ref[...]
    y = h + m
    mu = jnp.mean(y, axis=-1, keepdims=True)
    yc = y - mu
    var = jnp.mean(yc * yc, axis=-1, keepdims=True)
    out_ref[...] = yc * lax.rsqrt(var + 1e-5) * g_ref[...] + b_ref[...]


def _node(h, agg2, w1a, w1b, nb1, w2, nb2, g, b):
    n, d = h.shape
    blk = 2000
    wspec = lambda shape: pl.BlockSpec(shape, lambda i: tuple(0 for _ in shape))
    return pl.pallas_call(
        _node_body,
        grid=(n // blk,),
        in_specs=[
            pl.BlockSpec((blk, d), lambda i: (i, 0)),
            pl.BlockSpec((2, blk, 128), lambda i: (0, i, 0)),
            wspec((d, 2 * d)), wspec((64, 2 * d)), wspec((1, 2 * d)),
            wspec((2 * d, d)), wspec((1, d)),
            wspec((1, d)), wspec((1, d)),
        ],
        out_specs=pl.BlockSpec((blk, d), lambda i: (i, 0)),
        out_shape=jax.ShapeDtypeStruct((n, d), F32),
    )(h, agg2, w1a, w1b, nb1, w2, nb2, g, b)


def kernel(H, edge_index, dist2, delta, edge_struct,
           eW1, eb1, eW2, eb2, gW1, gb1, gW2, gb2,
           nW1, nb1, nW2, nb2, ln_g, ln_b):
    bz, n, d = H.shape
    e = edge_index.shape[1]
    d_struct = edge_struct.shape[-1]
    assert bz == 1 and e % _NW == 0 and d == 128

    h0 = H.reshape(n, d)
    ii = edge_index[0]
    jj = edge_index[1]
    ft = jnp.concatenate([dist2.reshape(1, e), delta.reshape(1, e),
                          edge_struct.reshape(e, d_struct).T], axis=0)

    wi = eW1[0:d]
    wj = eW1[d:2 * d]
    w1g = eW1[2 * d:]

    a, bm = _proj(h0, wi, wj)

    ns = 2
    es = e // ns
    emsgs, ii_parts = [], []
    for kk in range(ns):
        sl = slice(kk * es, (kk + 1) * es)
        ii_parts.append(ii[sl])
        gk = _gather(a, bm, ii_parts[kk], jj[sl])
        emsgs.append(_edge(gk, ft, kk * es, w1g,
                           eb1.reshape(1, -1), eW2, eb2.reshape(1, -1),
                           gW1, gb1.reshape(1, -1), gW2.reshape(1, -1),
                           gb2.reshape(1, 1)))
    zero = jnp.zeros((n, 64), F32)
    agg2 = _scatter(emsgs, ii_parts, zero).reshape(_NC, n, 128)
    out = _node(h0, agg2,
                nW1[0:d], nW1[d:], nb1.reshape(1, -1),
                nW2, nb2.reshape(1, -1),
                ln_g.reshape(1, -1), ln_b.reshape(1, -1))
    return out.reshape(bz, n, d)


# speedup vs baseline: 3.4163x; 1.0877x over previous
import functools

import jax
import jax.numpy as jnp
from jax import lax
from jax.experimental import pallas as pl
from jax.experimental.pallas import tpu as pltpu
from jax.experimental.pallas import tpu_sc as plsc

F32 = jnp.float32

_NC, _NS = 2, 16
_NW = _NC * _NS
_CH = 128


def _sigmoid(x):
    return 1.0 / (1.0 + jnp.exp(-x))


def _silu(x):
    return x * _sigmoid(x)


def _proj_body(h_ref, wi_ref, wj_ref, a_ref, b_ref):
    h = h_ref[...]
    a_ref[...] = jnp.dot(h, wi_ref[...], preferred_element_type=F32)
    b_ref[...] = jnp.dot(h, wj_ref[...], preferred_element_type=F32)


def _proj(h, wi, wj):
    n, d = h.shape
    blk = 2000
    return pl.pallas_call(
        _proj_body,
        grid=(n // blk,),
        in_specs=[
            pl.BlockSpec((blk, d), lambda i: (i, 0)),
            pl.BlockSpec((d, 64), lambda i: (0, 0)),
            pl.BlockSpec((d, 64), lambda i: (0, 0)),
        ],
        out_specs=[
            pl.BlockSpec((blk, 64), lambda i: (i, 0)),
            pl.BlockSpec((blk, 64), lambda i: (i, 0)),
        ],
        out_shape=[
            jax.ShapeDtypeStruct((n, 64), F32),
            jax.ShapeDtypeStruct((n, 64), F32),
        ],
    )(h, wi, wj)


def _pipe_loop(nfull, fire, wait, proc):
    for b in range(min(2, nfull)):
        fire(b, b)

    def pair(p, _):
        for b in range(2):
            cidx = 2 * p + b
            wait(cidx, b)
            proc(cidx, b)

            @pl.when(cidx + 2 < nfull)
            def _():
                fire(cidx + 2, b)
        return 0

    lax.fori_loop(0, nfull // 2, pair, 0)
    if nfull % 2:
        cidx = nfull - 1
        wait(cidx, cidx % 2)
        proc(cidx, cidx % 2)


def _gather_body(e, e_off, a_hbm, b_hbm, eidx_hbm, g_hbm,
                 ii_v, jj_v, rows_a, rows_b, sga0, sga1, sgb0, sgb1):
    c = lax.axis_index("c")
    s = lax.axis_index("s")
    wid = s * _NC + c
    ep = e // _NW
    nfull = ep // _CH
    tail = ep - nfull * _CH
    base = pl.multiple_of(wid * ep, _CH)
    sga = (sga0, sga1)
    sgb = (sgb0, sgb1)

    pltpu.sync_copy(eidx_hbm.at[0, pl.ds(e_off + base, ep)], ii_v)
    pltpu.sync_copy(eidx_hbm.at[1, pl.ds(e_off + base, ep)], jj_v)

    def fire(cidx, b):
        off = pl.multiple_of(cidx * _CH, _CH)
        pltpu.async_copy(a_hbm.at[ii_v.at[pl.ds(off, _CH)]], rows_a.at[b], sga[b])
        pltpu.async_copy(b_hbm.at[jj_v.at[pl.ds(off, _CH)]], rows_b.at[b], sgb[b])

    def wait(cidx, b):
        off = pl.multiple_of(cidx * _CH, _CH)
        pltpu.make_async_copy(a_hbm.at[ii_v.at[pl.ds(off, _CH)]],
                              rows_a.at[b], sga[b]).wait()
        pltpu.make_async_copy(b_hbm.at[jj_v.at[pl.ds(off, _CH)]],
                              rows_b.at[b], sgb[b]).wait()

    def proc(cidx, b):
        off = pl.multiple_of(cidx * _CH, _CH)
        pltpu.sync_copy(rows_a.at[b],
                        g_hbm.at[pl.ds(base + off, _CH), pl.ds(0, 64)])
        pltpu.sync_copy(rows_b.at[b],
                        g_hbm.at[pl.ds(base + off, _CH), pl.ds(64, 64)])

    _pipe_loop(nfull, fire, wait, proc)

    if tail:
        off = nfull * _CH
        ca = pltpu.async_copy(a_hbm.at[ii_v.at[pl.ds(off, tail)]],
                              rows_a.at[0, pl.ds(0, tail)], sga0)
        cb = pltpu.async_copy(b_hbm.at[jj_v.at[pl.ds(off, tail)]],
                              rows_b.at[0, pl.ds(0, tail)], sgb0)
        ca.wait()
        cb.wait()
        pltpu.sync_copy(rows_a.at[0, pl.ds(0, tail)],
                        g_hbm.at[pl.ds(base + off, tail), pl.ds(0, 64)])
        pltpu.sync_copy(rows_b.at[0, pl.ds(0, tail)],
                        g_hbm.at[pl.ds(base + off, tail), pl.ds(64, 64)])


def _gather(a, bm, eidx, e, e_off):
    n, d = a.shape
    ep = e // _NW
    mesh = plsc.VectorSubcoreMesh(core_axis_name="c", subcore_axis_name="s",
                                  num_cores=_NC, num_subcores=_NS)
    k = pl.kernel(
        functools.partial(_gather_body, e, e_off),
        mesh=mesh,
        compiler_params=pltpu.CompilerParams(use_tc_tiling_on_sc=False),
        out_type=jax.ShapeDtypeStruct((e, 2 * d), F32),
        scratch_types=[
            pltpu.VMEM((ep,), jnp.int32),
            pltpu.VMEM((ep,), jnp.int32),
            pltpu.VMEM((2, _CH, d), F32),
            pltpu.VMEM((2, _CH, d), F32),
            pltpu.SemaphoreType.DMA,
            pltpu.SemaphoreType.DMA,
            pltpu.SemaphoreType.DMA,
            pltpu.SemaphoreType.DMA,
        ],
    )
    return k(a, bm, eidx)


def _edge_body(g_ref, ft_ref,
               w1g_ref, eb1_ref, ew2_ref, eb2_ref,
               gw1_ref, gb1_ref, gw2_ref, gb2_ref, out_ref):
    g = g_ref[...]
    x = g[:, 0:64] + g[:, 64:128]
    ft = ft_ref[...]
    pre = (x
           + jax.lax.dot_general(ft, w1g_ref[...], (((0,), (0,)), ((), ())),
                                 preferred_element_type=F32)
           + eb1_ref[...])
    h = _silu(pre)
    e = _silu(jnp.dot(h, ew2_ref[...], preferred_element_type=F32) + eb2_ref[...])
    g1 = (jax.lax.dot_general(ft, gw1_ref[...], (((0,), (0,)), ((), ())),
                              preferred_element_type=F32)
          + gb1_ref[...])
    gh = _silu(g1)
    glogit = jnp.sum(gh * gw2_ref[...], axis=-1, keepdims=True) + gb2_ref[...]
    msg = e * _sigmoid(glogit)
    out_ref[...] = jnp.concatenate([msg, jnp.zeros_like(msg)], axis=1)


def _edge(g, ft, ft_off, w1g, eb1, ew2, eb2, gw1, gb1, gw2, gb2):
    e = g.shape[0]
    blk = 6400
    off_blocks = ft_off // blk
    wspec = lambda shape: pl.BlockSpec(shape, lambda i: tuple(0 for _ in shape))
    return pl.pallas_call(
        _edge_body,
        grid=(e // blk,),
        in_specs=[
            pl.BlockSpec((blk, 128), lambda i: (i, 0)),
            pl.BlockSpec((10, blk), lambda i: (0, i + off_blocks)),
            wspec((10, 64)), wspec((1, 64)), wspec((64, 64)), wspec((1, 64)),
            wspec((10, 32)), wspec((1, 32)), wspec((1, 32)), wspec((1, 1)),
        ],
        out_specs=pl.BlockSpec((blk, 128), lambda i: (i, 0)),
        out_shape=jax.ShapeDtypeStruct((e, 128), F32),
    )(g, ft, w1g, eb1, ew2, eb2, gw1, gb1, gw2, gb2)


def _scatter_body(e, e_off, n, msg_hbm, eidx_hbm, zero_hbm, out_hbm,
                  idx_v, idx_t, rows_v, rows_t, agg_sh, sr0, sr1, si0, si1):
    c = lax.axis_index("c")
    s = lax.axis_index("s")
    wid = s * _NC + c
    ep = e // _NW
    nfull = ep // _CH
    tail = ep - nfull * _CH
    base = pl.multiple_of(wid * ep, _CH)
    npart = n // _NS
    sr = (sr0, sr1)
    si = (si0, si1)

    pltpu.sync_copy(zero_hbm.at[pl.ds(s * npart, npart)],
                    agg_sh.at[pl.ds(s * npart, npart)])
    plsc.subcore_barrier()

    def fire(cidx, b):
        off = pl.multiple_of(cidx * _CH, _CH)
        pltpu.async_copy(msg_hbm.at[pl.ds(base + off, _CH), pl.ds(0, 64)],
                         rows_v.at[b], sr[b])
        pltpu.async_copy(eidx_hbm.at[0, pl.ds(e_off + base + off, _CH)],
                         idx_v.at[b], si[b])

    def wait(cidx, b):
        off = pl.multiple_of(cidx * _CH, _CH)
        pltpu.make_async_copy(msg_hbm.at[pl.ds(base + off, _CH), pl.ds(0, 64)],
                              rows_v.at[b], sr[b]).wait()
        pltpu.make_async_copy(eidx_hbm.at[0, pl.ds(e_off + base + off, _CH)],
                              idx_v.at[b], si[b]).wait()

    def proc(cidx, b):
        pltpu.sync_copy(rows_v.at[b], agg_sh.at[idx_v.at[b]], add=True)

    _pipe_loop(nfull, fire, wait, proc)

    if tail:
        off = nfull * _CH
        pltpu.sync_copy(msg_hbm.at[pl.ds(base + off, tail), pl.ds(0, 64)],
                        rows_t)
        pltpu.sync_copy(eidx_hbm.at[0, pl.ds(e_off + base + off, tail)], idx_t)
        pltpu.sync_copy(rows_t, agg_sh.at[idx_t], add=True)

    plsc.subcore_barrier()
    pltpu.sync_copy(agg_sh.at[pl.ds(s * npart, npart)],
                    out_hbm.at[pl.ds(c * n + s * npart, npart), pl.ds(0, 64)])


def _scatter(msg, eidx, e_off, zero):
    e = msg.shape[0]
    n = zero.shape[0]
    ep = e // _NW
    tail = ep - (ep // _CH) * _CH
    mesh = plsc.VectorSubcoreMesh(core_axis_name="c", subcore_axis_name="s",
                                  num_cores=_NC, num_subcores=_NS)
    k = pl.kernel(
        functools.partial(_scatter_body, e, e_off, n),
        mesh=mesh,
        compiler_params=pltpu.CompilerParams(use_tc_tiling_on_sc=False),
        out_type=jax.ShapeDtypeStruct((_NC * n, 128), F32),
        scratch_types=[
            pltpu.VMEM((2, _CH), jnp.int32),
            pltpu.VMEM((max(tail, 8),), jnp.int32),
            pltpu.VMEM((2, _CH, 64), F32),
            pltpu.VMEM((max(tail, 8), 64), F32),
            pltpu.VMEM_SHARED((n, 64), F32),
            pltpu.SemaphoreType.DMA,
            pltpu.SemaphoreType.DMA,
            pltpu.SemaphoreType.DMA,
            pltpu.SemaphoreType.DMA,
        ],
    )
    return k(msg, eidx, zero)


def _node_body(na, h_ref, *refs):
    (w1a_ref, w1b_ref, nb1_ref, w2_ref, nb2_ref, g_ref, b_ref,
     out_ref) = refs[na:]
    h = h_ref[...]
    atot = refs[0][0] + refs[0][1]
    for r in refs[1:na]:
        atot = atot + r[0] + r[1]
    agg = atot[:, 0:64]
    m1 = (jnp.dot(h, w1a_ref[...], preferred_element_type=F32)
          + jnp.dot(agg, w1b_ref[...], preferred_element_type=F32)
          + nb1_ref[...])
    hm = _silu(m1)
    m = jnp.dot(hm, w2_ref[...], preferred_element_type=F32) + nb2_ref[...]
    y = h + m
    mu = jnp.mean(y, axis=-1, keepdims=True)
    yc = y - mu
    var = jnp.mean(yc * yc, axis=-1, keepdims=True)
    out_ref[...] = yc * lax.rsqrt(var + 1e-5) * g_ref[...] + b_ref[...]


def _node(h, aggs, w1a, w1b, nb1, w2, nb2, g, b):
    n, d = h.shape
    blk = 2000
    wspec = lambda shape: pl.BlockSpec(shape, lambda i: tuple(0 for _ in shape))
    return pl.pallas_call(
        functools.partial(_node_body, len(aggs)),
        grid=(n // blk,),
        in_specs=[
            pl.BlockSpec((blk, d), lambda i: (i, 0)),
            *[pl.BlockSpec((2, blk, 128), lambda i: (0, i, 0)) for _ in aggs],
            wspec((d, 2 * d)), wspec((64, 2 * d)), wspec((1, 2 * d)),
            wspec((2 * d, d)), wspec((1, d)),
            wspec((1, d)), wspec((1, d)),
        ],
        out_specs=pl.BlockSpec((blk, d), lambda i: (i, 0)),
        out_shape=jax.ShapeDtypeStruct((n, d), F32),
    )(h, *aggs, w1a, w1b, nb1, w2, nb2, g, b)


def kernel(H, edge_index, dist2, delta, edge_struct,
           eW1, eb1, eW2, eb2, gW1, gb1, gW2, gb2,
           nW1, nb1, nW2, nb2, ln_g, ln_b):
    bz, n, d = H.shape
    e = edge_index.shape[1]
    d_struct = edge_struct.shape[-1]
    assert bz == 1 and e % _NW == 0 and d == 128

    h0 = H.reshape(n, d)
    ft = jnp.concatenate([dist2.reshape(1, e), delta.reshape(1, e),
                          edge_struct.reshape(e, d_struct).T], axis=0)

    wi = eW1[0:d]
    wj = eW1[d:2 * d]
    w1g = eW1[2 * d:]

    a, bm = _proj(h0, wi, wj)

    ns = 2
    es = e // ns
    zero = jnp.zeros((n, 64), F32)
    aggs = []
    for kk in range(ns):
        gk = _gather(a, bm, edge_index, es, kk * es)
        ek = _edge(gk, ft, kk * es, w1g,
                   eb1.reshape(1, -1), eW2, eb2.reshape(1, -1),
                   gW1, gb1.reshape(1, -1), gW2.reshape(1, -1),
                   gb2.reshape(1, 1))
        aggs.append(_scatter(ek, edge_index, kk * es, zero).reshape(_NC, n, 128))
    out = _node(h0, aggs,
                nW1[0:d], nW1[d:], nb1.reshape(1, -1),
                nW2, nb2.reshape(1, -1),
                ln_g.reshape(1, -1), ln_b.reshape(1, -1))
    return out.reshape(bz, n, d)


# speedup vs baseline: 3.5193x; 1.0301x over previous
import functools

import jax
import jax.numpy as jnp
from jax import lax
from jax.experimental import pallas as pl
from jax.experimental.pallas import tpu as pltpu
from jax.experimental.pallas import tpu_sc as plsc

F32 = jnp.float32

_NC, _NS = 2, 16
_NW = _NC * _NS
_CH = 128


def _sigmoid(x):
    return 1.0 / (1.0 + jnp.exp(-x))


def _silu(x):
    return x * _sigmoid(x)


def _proj_body(h_ref, wi_ref, wj_ref, a_ref, b_ref):
    h = h_ref[...]
    a_ref[...] = jnp.dot(h, wi_ref[...], preferred_element_type=F32)
    b_ref[...] = jnp.dot(h, wj_ref[...], preferred_element_type=F32)


def _proj(h, wi, wj):
    n, d = h.shape
    blk = 2000
    return pl.pallas_call(
        _proj_body,
        grid=(n // blk,),
        in_specs=[
            pl.BlockSpec((blk, d), lambda i: (i, 0)),
            pl.BlockSpec((d, 64), lambda i: (0, 0)),
            pl.BlockSpec((d, 64), lambda i: (0, 0)),
        ],
        out_specs=[
            pl.BlockSpec((blk, 64), lambda i: (i, 0)),
            pl.BlockSpec((blk, 64), lambda i: (i, 0)),
        ],
        out_shape=[
            jax.ShapeDtypeStruct((n, 64), F32),
            jax.ShapeDtypeStruct((n, 64), F32),
        ],
    )(h, wi, wj)


def _pipe_loop(nfull, fire, wait, proc):
    for b in range(min(2, nfull)):
        fire(b, b)

    def pair(p, _):
        for b in range(2):
            cidx = 2 * p + b
            wait(cidx, b)
            proc(cidx, b)

            @pl.when(cidx + 2 < nfull)
            def _():
                fire(cidx + 2, b)
        return 0

    lax.fori_loop(0, nfull // 2, pair, 0)
    if nfull % 2:
        cidx = nfull - 1
        wait(cidx, cidx % 2)
        proc(cidx, cidx % 2)


def _gather_body(e, e_off, a_hbm, b_hbm, eidx_hbm, g_hbm,
                 ii_v, jj_v, rows_a, rows_b, sga0, sga1, sgb0, sgb1):
    c = lax.axis_index("c")
    s = lax.axis_index("s")
    wid = s * _NC + c
    ep = e // _NW
    nfull = ep // _CH
    tail = ep - nfull * _CH
    base = pl.multiple_of(wid * ep, _CH)
    sga = (sga0, sga1)
    sgb = (sgb0, sgb1)

    pltpu.sync_copy(eidx_hbm.at[0, pl.ds(e_off + base, ep)], ii_v)
    pltpu.sync_copy(eidx_hbm.at[1, pl.ds(e_off + base, ep)], jj_v)

    def fire(cidx, b):
        off = pl.multiple_of(cidx * _CH, _CH)
        pltpu.async_copy(a_hbm.at[ii_v.at[pl.ds(off, _CH)]], rows_a.at[b], sga[b])
        pltpu.async_copy(b_hbm.at[jj_v.at[pl.ds(off, _CH)]], rows_b.at[b], sgb[b])

    def wait(cidx, b):
        off = pl.multiple_of(cidx * _CH, _CH)
        pltpu.make_async_copy(a_hbm.at[ii_v.at[pl.ds(off, _CH)]],
                              rows_a.at[b], sga[b]).wait()
        pltpu.make_async_copy(b_hbm.at[jj_v.at[pl.ds(off, _CH)]],
                              rows_b.at[b], sgb[b]).wait()

    def proc(cidx, b):
        off = pl.multiple_of(cidx * _CH, _CH)
        pltpu.sync_copy(rows_a.at[b],
                        g_hbm.at[pl.ds(base + off, _CH), pl.ds(0, 64)])
        pltpu.sync_copy(rows_b.at[b],
                        g_hbm.at[pl.ds(base + off, _CH), pl.ds(64, 64)])

    _pipe_loop(nfull, fire, wait, proc)

    if tail:
        off = nfull * _CH
        ca = pltpu.async_copy(a_hbm.at[ii_v.at[pl.ds(off, tail)]],
                              rows_a.at[0, pl.ds(0, tail)], sga0)
        cb = pltpu.async_copy(b_hbm.at[jj_v.at[pl.ds(off, tail)]],
                              rows_b.at[0, pl.ds(0, tail)], sgb0)
        ca.wait()
        cb.wait()
        pltpu.sync_copy(rows_a.at[0, pl.ds(0, tail)],
                        g_hbm.at[pl.ds(base + off, tail), pl.ds(0, 64)])
        pltpu.sync_copy(rows_b.at[0, pl.ds(0, tail)],
                        g_hbm.at[pl.ds(base + off, tail), pl.ds(64, 64)])


def _gather(a, bm, eidx, e, e_off):
    n, d = a.shape
    ep = e // _NW
    mesh = plsc.VectorSubcoreMesh(core_axis_name="c", subcore_axis_name="s",
                                  num_cores=_NC, num_subcores=_NS)
    k = pl.kernel(
        functools.partial(_gather_body, e, e_off),
        mesh=mesh,
        compiler_params=pltpu.CompilerParams(use_tc_tiling_on_sc=False),
        out_type=jax.ShapeDtypeStruct((e, 2 * d), F32),
        scratch_types=[
            pltpu.VMEM((ep,), jnp.int32),
            pltpu.VMEM((ep,), jnp.int32),
            pltpu.VMEM((2, _CH, d), F32),
            pltpu.VMEM((2, _CH, d), F32),
            pltpu.SemaphoreType.DMA,
            pltpu.SemaphoreType.DMA,
            pltpu.SemaphoreType.DMA,
            pltpu.SemaphoreType.DMA,
        ],
    )
    return k(a, bm, eidx)


def _edge_body(g_ref, ft_ref,
               w1g_ref, eb1_ref, ew2_ref, eb2_ref,
               gw1_ref, gb1_ref, gw2_ref, gb2_ref, out_ref):
    g = g_ref[...]
    x = g[:, 0:64] + g[:, 64:128]
    ft = ft_ref[...]
    pre = (x
           + jax.lax.dot_general(ft, w1g_ref[...], (((0,), (0,)), ((), ())),
                                 preferred_element_type=F32)
           + eb1_ref[...])
    h = _silu(pre)
    e = _silu(jnp.dot(h, ew2_ref[...], preferred_element_type=F32) + eb2_ref[...])
    g1 = (jax.lax.dot_general(ft, gw1_ref[...], (((0,), (0,)), ((), ())),
                              preferred_element_type=F32)
          + gb1_ref[...])
    gh = _silu(g1)
    glogit = jnp.sum(gh * gw2_ref[...], axis=-1, keepdims=True) + gb2_ref[...]
    msg = e * _sigmoid(glogit)
    out_ref[...] = jnp.concatenate([msg, jnp.zeros_like(msg)], axis=1)


def _edge(g, ft, ft_off, w1g, eb1, ew2, eb2, gw1, gb1, gw2, gb2):
    e = g.shape[0]
    blk = 6400
    off_blocks = ft_off // blk
    wspec = lambda shape: pl.BlockSpec(shape, lambda i: tuple(0 for _ in shape))
    return pl.pallas_call(
        _edge_body,
        grid=(e // blk,),
        in_specs=[
            pl.BlockSpec((blk, 128), lambda i: (i, 0)),
            pl.BlockSpec((10, blk), lambda i: (0, i + off_blocks)),
            wspec((10, 64)), wspec((1, 64)), wspec((64, 64)), wspec((1, 64)),
            wspec((10, 32)), wspec((1, 32)), wspec((1, 32)), wspec((1, 1)),
        ],
        out_specs=pl.BlockSpec((blk, 128), lambda i: (i, 0)),
        out_shape=jax.ShapeDtypeStruct((e, 128), F32),
    )(g, ft, w1g, eb1, ew2, eb2, gw1, gb1, gw2, gb2)


def _scatter_body(e, e_off, n, msg_hbm, eidx_hbm, zero_hbm, out_hbm,
                  idx_v, idx_t, rows_v, rows_t, agg_sh, sr0, sr1, si0, si1):
    c = lax.axis_index("c")
    s = lax.axis_index("s")
    wid = s * _NC + c
    ep = e // _NW
    nfull = ep // _CH
    tail = ep - nfull * _CH
    base = pl.multiple_of(wid * ep, _CH)
    npart = n // _NS
    sr = (sr0, sr1)
    si = (si0, si1)

    pltpu.sync_copy(zero_hbm.at[pl.ds(s * npart, npart)],
                    agg_sh.at[pl.ds(s * npart, npart)])
    plsc.subcore_barrier()

    def fire(cidx, b):
        off = pl.multiple_of(cidx * _CH, _CH)
        pltpu.async_copy(msg_hbm.at[pl.ds(base + off, _CH), pl.ds(0, 64)],
                         rows_v.at[b], sr[b])
        pltpu.async_copy(eidx_hbm.at[0, pl.ds(e_off + base + off, _CH)],
                         idx_v.at[b], si[b])

    def wait(cidx, b):
        off = pl.multiple_of(cidx * _CH, _CH)
        pltpu.make_async_copy(msg_hbm.at[pl.ds(base + off, _CH), pl.ds(0, 64)],
                              rows_v.at[b], sr[b]).wait()
        pltpu.make_async_copy(eidx_hbm.at[0, pl.ds(e_off + base + off, _CH)],
                              idx_v.at[b], si[b]).wait()

    def proc(cidx, b):
        pltpu.sync_copy(rows_v.at[b], agg_sh.at[idx_v.at[b]], add=True)

    _pipe_loop(nfull, fire, wait, proc)

    if tail:
        off = nfull * _CH
        pltpu.sync_copy(msg_hbm.at[pl.ds(base + off, tail), pl.ds(0, 64)],
                        rows_t)
        pltpu.sync_copy(eidx_hbm.at[0, pl.ds(e_off + base + off, tail)], idx_t)
        pltpu.sync_copy(rows_t, agg_sh.at[idx_t], add=True)

    plsc.subcore_barrier()
    pltpu.sync_copy(agg_sh.at[pl.ds(s * npart, npart)],
                    out_hbm.at[pl.ds(c * n + s * npart, npart), pl.ds(0, 64)])


def _scatter(msg, eidx, e_off, zero):
    e = msg.shape[0]
    n = zero.shape[0]
    ep = e // _NW
    tail = ep - (ep // _CH) * _CH
    mesh = plsc.VectorSubcoreMesh(core_axis_name="c", subcore_axis_name="s",
                                  num_cores=_NC, num_subcores=_NS)
    k = pl.kernel(
        functools.partial(_scatter_body, e, e_off, n),
        mesh=mesh,
        compiler_params=pltpu.CompilerParams(use_tc_tiling_on_sc=False),
        out_type=jax.ShapeDtypeStruct((_NC * n, 128), F32),
        scratch_types=[
            pltpu.VMEM((2, _CH), jnp.int32),
            pltpu.VMEM((max(tail, 8),), jnp.int32),
            pltpu.VMEM((2, _CH, 64), F32),
            pltpu.VMEM((max(tail, 8), 64), F32),
            pltpu.VMEM_SHARED((n, 64), F32),
            pltpu.SemaphoreType.DMA,
            pltpu.SemaphoreType.DMA,
            pltpu.SemaphoreType.DMA,
            pltpu.SemaphoreType.DMA,
        ],
    )
    return k(msg, eidx, zero)


def _node_body(na, h_ref, *refs):
    (w1a_ref, w1b_ref, nb1_ref, w2_ref, nb2_ref, g_ref, b_ref,
     out_ref) = refs[na:]
    h = h_ref[...]
    atot = refs[0][0] + refs[0][1]
    for r in refs[1:na]:
        atot = atot + r[0] + r[1]
    agg = atot[:, 0:64]
    m1 = (jnp.dot(h, w1a_ref[...], preferred_element_type=F32)
          + jnp.dot(agg, w1b_ref[...], preferred_element_type=F32)
          + nb1_ref[...])
    hm = _silu(m1)
    m = jnp.dot(hm, w2_ref[...], preferred_element_type=F32) + nb2_ref[...]
    y = h + m
    mu = jnp.mean(y, axis=-1, keepdims=True)
    yc = y - mu
    var = jnp.mean(yc * yc, axis=-1, keepdims=True)
    out_ref[...] = yc * lax.rsqrt(var + 1e-5) * g_ref[...] + b_ref[...]


def _node(h, aggs, w1a, w1b, nb1, w2, nb2, g, b):
    n, d = h.shape
    blk = 2000
    wspec = lambda shape: pl.BlockSpec(shape, lambda i: tuple(0 for _ in shape))
    return pl.pallas_call(
        functools.partial(_node_body, len(aggs)),
        grid=(n // blk,),
        in_specs=[
            pl.BlockSpec((blk, d), lambda i: (i, 0)),
            *[pl.BlockSpec((2, blk, 128), lambda i: (0, i, 0)) for _ in aggs],
            wspec((d, 2 * d)), wspec((64, 2 * d)), wspec((1, 2 * d)),
            wspec((2 * d, d)), wspec((1, d)),
            wspec((1, d)), wspec((1, d)),
        ],
        out_specs=pl.BlockSpec((blk, d), lambda i: (i, 0)),
        out_shape=jax.ShapeDtypeStruct((n, d), F32),
    )(h, *aggs, w1a, w1b, nb1, w2, nb2, g, b)


def kernel(H, edge_index, dist2, delta, edge_struct,
           eW1, eb1, eW2, eb2, gW1, gb1, gW2, gb2,
           nW1, nb1, nW2, nb2, ln_g, ln_b):
    bz, n, d = H.shape
    e = edge_index.shape[1]
    d_struct = edge_struct.shape[-1]
    assert bz == 1 and e % _NW == 0 and d == 128

    h0 = H.reshape(n, d)
    ft = jnp.concatenate([dist2.reshape(1, e), delta.reshape(1, e),
                          edge_struct.reshape(e, d_struct).T], axis=0)

    wi = eW1[0:d]
    wj = eW1[d:2 * d]
    w1g = eW1[2 * d:]

    a, bm = _proj(h0, wi, wj)

    ns = 5
    es = e // ns
    zero = jnp.zeros((n, 64), F32)
    aggs = []
    for kk in range(ns):
        gk = _gather(a, bm, edge_index, es, kk * es)
        ek = _edge(gk, ft, kk * es, w1g,
                   eb1.reshape(1, -1), eW2, eb2.reshape(1, -1),
                   gW1, gb1.reshape(1, -1), gW2.reshape(1, -1),
                   gb2.reshape(1, 1))
        aggs.append(_scatter(ek, edge_index, kk * es, zero).reshape(_NC, n, 128))
    out = _node(h0, aggs,
                nW1[0:d], nW1[d:], nb1.reshape(1, -1),
                nW2, nb2.reshape(1, -1),
                ln_g.reshape(1, -1), ln_b.reshape(1, -1))
    return out.reshape(bz, n, d)
